# baseline (speedup 1.0000x reference)
import functools

import jax
import jax.numpy as jnp
from jax.experimental import pallas as pl
from jax.experimental.pallas import tpu as pltpu
from jax.experimental.pallas import tpu_sc as plsc

_N = 2048
_D = 768
_KB = 2048
_NSC = 32
_GR = _N // _NSC


def _dist_argmin_body(kb, r_ref, cb_ref, a2_ref, b2_ref, idx_ref, ssq_ref, best_ref):
    k = pl.program_id(1)
    ab = jax.lax.dot_general(
        r_ref[...], cb_ref[...],
        dimension_numbers=(((1,), (1,)), ((), ())),
        preferred_element_type=jnp.float32,
    )
    s = a2_ref[...] + b2_ref[...]
    d2 = jnp.maximum(s - 2.0 * ab, 0.0)
    m = jnp.min(d2, axis=1, keepdims=True)
    j = jax.lax.broadcasted_iota(jnp.int32, d2.shape, 1)
    lidx = jnp.min(jnp.where(d2 == m, j, jnp.int32(2**30)), axis=1, keepdims=True)
    gidx = lidx + k * kb

    @pl.when(k == 0)
    def _():
        best_ref[...] = m
        idx_ref[...] = gidx
        ssq_ref[...] = jnp.full(ssq_ref.shape, jnp.sum(a2_ref[...]), jnp.float32)

    @pl.when(k > 0)
    def _():
        better = m < best_ref[...]
        idx_ref[...] = jnp.where(better, gidx, idx_ref[...])
        best_ref[...] = jnp.where(better, m, best_ref[...])


@functools.partial(jax.jit, static_argnames=("kk",))
def _dist_argmin(r, cb, a2, b2, kk):
    n = r.shape[0]
    nt = n // 2
    kb = min(kk, _KB)
    return pl.pallas_call(
        functools.partial(_dist_argmin_body, kb),
        grid=(2, kk // kb),
        in_specs=[
            pl.BlockSpec((nt, _D), lambda i, k: (i, 0)),
            pl.BlockSpec((kb, _D), lambda i, k: (k, 0)),
            pl.BlockSpec((nt, 1), lambda i, k: (i, 0)),
            pl.BlockSpec((1, kb), lambda i, k: (0, k)),
        ],
        out_specs=[
            pl.BlockSpec((nt, 1), lambda i, k: (i, 0)),
            pl.BlockSpec((8, 128), lambda i, k: (i, 0)),
        ],
        out_shape=[
            jax.ShapeDtypeStruct((n, 1), jnp.int32),
            jax.ShapeDtypeStruct((16, 128), jnp.float32),
        ],
        scratch_shapes=[pltpu.VMEM((nt, 1), jnp.float32)],
        compiler_params=pltpu.CompilerParams(
            dimension_semantics=("parallel", "arbitrary"),
        ),
    )(r, cb, a2, b2)


def _sc_gather(cb, idx):
    mesh = plsc.VectorSubcoreMesh(core_axis_name="core", subcore_axis_name="subcore")
    n = idx.shape[0] * 128
    gr = n // _NSC
    share = 128 // gr

    @pl.kernel(
        out_type=jax.ShapeDtypeStruct((n, _D), jnp.float32),
        mesh=mesh,
        scratch_types=[
            pltpu.VMEM((gr, _D), jnp.float32),
            pltpu.VMEM((1, 128), jnp.int32),
        ],
    )
    def kern(cb_hbm, i_hbm, o_hbm, qbuf, ibuf):
        u = jax.lax.axis_index("core") * 16 + jax.lax.axis_index("subcore")
        pltpu.sync_copy(i_hbm.at[pl.ds(u // share, 1)], ibuf)
        off = (u % share) * gr
        pltpu.sync_copy(cb_hbm.at[ibuf.at[0, pl.ds(off, gr)]], qbuf)
        pltpu.sync_copy(qbuf, o_hbm.at[pl.ds(u * gr, gr)])

    return kern(cb, idx)


def kernel(x, codebook_0, codebook_1, codebook_2, codebook_3):
    codebooks = [codebook_0, codebook_1, codebook_2, codebook_3]
    b, t, d = x.shape
    x2d = x.reshape(-1, d)
    nh = _N // 2
    halves = [x2d[:nh], x2d[nh:]]
    a2s = [jnp.sum(h * h, axis=1, keepdims=True) for h in halves]
    all_indices = []
    commit_ssq = []
    for cb in codebooks:
        b2 = jnp.sum(cb * cb, axis=1)[None, :]
        layer_idx = []
        layer_ssq = []
        for h in range(2):
            idx, ssq = _dist_argmin(halves[h], cb, a2s[h], b2, cb.shape[0])
            layer_ssq.append(ssq[0, 0] + ssq[8, 0])
            q = _sc_gather(cb, idx.reshape(-1, 128))
            r = halves[h]
            q_st = r + (q - r)
            halves[h] = r - q_st
            a2s[h] = jnp.sum(halves[h] * halves[h], axis=1, keepdims=True)
            layer_idx.append(idx)
        commit_ssq.append(layer_ssq[0] + layer_ssq[1])
        all_indices.append(
            jnp.concatenate(layer_idx, axis=0).reshape(b, t))
    quantized = jnp.concatenate([x2d[:nh] - halves[0], x2d[nh:] - halves[1]],
                                axis=0)
    total_commit = (
        (commit_ssq[1] + commit_ssq[2] + commit_ssq[3]
         + jnp.sum(a2s[0]) + jnp.sum(a2s[1]))
        * (0.25 / (b * t * d))
    ).astype(jnp.float32)
    all_indices = jnp.stack(all_indices, axis=-1)
    return quantized.reshape(b, t, d), all_indices, total_commit


# speedup vs baseline: 1.2531x; 1.2531x over previous
import functools

import jax
import jax.numpy as jnp
from jax.experimental import pallas as pl
from jax.experimental.pallas import tpu as pltpu
from jax.experimental.pallas import tpu_sc as plsc

_N = 2048
_D = 768
_KB = 2048
_NSC = 32
_GR = _N // _NSC


def _dist_argmin_body(kb, r_ref, cb_ref, a2_ref, b2_ref, idx_ref, ssq_ref, best_ref):
    k = pl.program_id(1)
    ab = jax.lax.dot_general(
        r_ref[...], cb_ref[...],
        dimension_numbers=(((1,), (1,)), ((), ())),
        preferred_element_type=jnp.float32,
    )
    s = a2_ref[...] + b2_ref[...]
    d2 = jnp.maximum(s - 2.0 * ab, 0.0)
    m = jnp.min(d2, axis=1, keepdims=True)
    j = jax.lax.broadcasted_iota(jnp.int32, d2.shape, 1)
    lidx = jnp.min(jnp.where(d2 == m, j, jnp.int32(2**30)), axis=1, keepdims=True)
    gidx = lidx + k * kb

    @pl.when(k == 0)
    def _():
        best_ref[...] = m
        idx_ref[...] = gidx
        ssq_ref[...] = jnp.full(ssq_ref.shape, jnp.sum(a2_ref[...]), jnp.float32)

    @pl.when(k > 0)
    def _():
        better = m < best_ref[...]
        idx_ref[...] = jnp.where(better, gidx, idx_ref[...])
        best_ref[...] = jnp.where(better, m, best_ref[...])


@functools.partial(jax.jit, static_argnames=("kk",))
def _dist_argmin(r, cb, a2, b2, kk):
    nt = _N // 2
    kb = min(kk, _KB)
    return pl.pallas_call(
        functools.partial(_dist_argmin_body, kb),
        grid=(2, kk // kb),
        in_specs=[
            pl.BlockSpec((nt, _D), lambda i, k: (i, 0)),
            pl.BlockSpec((kb, _D), lambda i, k: (k, 0)),
            pl.BlockSpec((nt, 1), lambda i, k: (i, 0)),
            pl.BlockSpec((1, kb), lambda i, k: (0, k)),
        ],
        out_specs=[
            pl.BlockSpec((nt, 1), lambda i, k: (i, 0)),
            pl.BlockSpec((8, 128), lambda i, k: (i, 0)),
        ],
        out_shape=[
            jax.ShapeDtypeStruct((_N, 1), jnp.int32),
            jax.ShapeDtypeStruct((16, 128), jnp.float32),
        ],
        scratch_shapes=[pltpu.VMEM((nt, 1), jnp.float32)],
        compiler_params=pltpu.CompilerParams(
            dimension_semantics=("parallel", "arbitrary"),
        ),
    )(r, cb, a2, b2)


def _sc_gather(cb, idx):
    mesh = plsc.VectorSubcoreMesh(core_axis_name="core", subcore_axis_name="subcore")

    @pl.kernel(
        out_type=jax.ShapeDtypeStruct((_N, _D), jnp.float32),
        mesh=mesh,
        scratch_types=[
            pltpu.VMEM((_GR, _D), jnp.float32),
            pltpu.VMEM((1, 128), jnp.int32),
        ],
    )
    def kern(cb_hbm, i_hbm, o_hbm, qbuf, ibuf):
        u = jax.lax.axis_index("core") * 16 + jax.lax.axis_index("subcore")
        pltpu.sync_copy(i_hbm.at[pl.ds(u // 2, 1)], ibuf)
        off = (u % 2) * _GR
        pltpu.sync_copy(cb_hbm.at[ibuf.at[0, pl.ds(off, _GR)]], qbuf)
        pltpu.sync_copy(qbuf, o_hbm.at[pl.ds(u * _GR, _GR)])

    return kern(cb, idx)


def kernel(x, codebook_0, codebook_1, codebook_2, codebook_3):
    codebooks = [codebook_0, codebook_1, codebook_2, codebook_3]
    b, t, d = x.shape
    x2d = x.reshape(-1, d)
    residual = x2d
    a2 = jnp.sum(residual * residual, axis=1, keepdims=True)
    all_indices = []
    commit_ssq = []
    for cb in codebooks:
        b2 = jnp.sum(cb * cb, axis=1)[None, :]
        idx, ssq = _dist_argmin(residual, cb, a2, b2, cb.shape[0])
        commit_ssq.append(ssq[0, 0] + ssq[8, 0])
        q = _sc_gather(cb, idx.reshape(16, 128))
        q_st = residual + (q - residual)
        residual = residual - q_st
        a2 = jnp.sum(residual * residual, axis=1, keepdims=True)
        all_indices.append(idx.reshape(b, t))
    quantized = x2d - residual
    total_commit = (
        (commit_ssq[1] + commit_ssq[2] + commit_ssq[3] + jnp.sum(a2))
        * (0.25 / (b * t * d))
    ).astype(jnp.float32)
    all_indices = jnp.stack(all_indices, axis=-1)
    return quantized.reshape(b, t, d), all_indices, total_commit
